# baseline (speedup 1.0000x reference)
import jax
import jax.numpy as jnp
import numpy as _np
from jax.experimental import pallas as pl

N = 512
INPUT_DIM = 32
EMBED_DIM = 64
HIDDEN = 64
LATENT = 32
PERIODS = 3

_EPS = _np.asarray(
    jax.random.normal(jax.random.key(42), (N, LATENT), jnp.float32))

_B_ENT, _B_TIM, _B_ECZ, _B_ELZ, _B_ECH, _B_ELH = 0, 64, 128, 192, 256, 320
_B_MU, _B_LV, _B_DEC = 384, 416, 448
_B_DCZ, _B_DLZ, _B_DCH, _B_DLH, _B_ATT = 512, 544, 576, 608, 640


def _colsum_contract(a, b):
    return jax.lax.dot_general(
        a, b, (((0,), (0,)), ((), ())), preferred_element_type=jnp.float32)


def _mm(a, b):
    return jnp.dot(a, b, preferred_element_type=jnp.float32)


def _fwd_kernel(
    x_ref, ent_ref, tim_ref, eps_ref,
    ws_ref, as_ref,
    entW_ref, timW_ref, ezW_ref, ehW_ref, elzW_ref, elhW_ref,
    muW_ref, lvW_ref, decW_ref, dzW_ref, dhW_ref, dlzW_ref, dlhW_ref,
    bias_ref,
    recon_ref, mu_ref, lv_ref, w_ref, a_ref,
):
    def bias(off, width=HIDDEN):
        return bias_ref[0:1, off:off + width]

    ri = jax.lax.broadcasted_iota(jnp.int32, (N, N), 0)
    ci = jax.lax.broadcasted_iota(jnp.int32, (N, N), 1)
    W = jnp.where(ri == ci, 0.0, jax.nn.sigmoid(ws_ref[...]))
    A = jax.nn.sigmoid(as_ref[...])
    w_ref[...] = W
    a_ref[...] = A
    S = W + A

    ones = jnp.ones((N, 1), jnp.float32)
    deg = _colsum_contract(S, ones) + 1.0
    dinv = jax.lax.rsqrt(deg)
    dinv2 = dinv * dinv

    probs = jax.nn.softmax(bias(_B_ATT, PERIODS), axis=-1)

    ent_all = jax.nn.relu(
        _mm(jnp.reshape(ent_ref[...], (PERIODS * N, EMBED_DIM)),
            entW_ref[...]) + bias(_B_ENT))
    tim_all = jax.nn.relu(
        _mm(jnp.reshape(tim_ref[...], (PERIODS * N, EMBED_DIM)),
            timW_ref[...]) + bias(_B_TIM))
    h_all = jnp.concatenate(
        [jnp.reshape(x_ref[...], (PERIODS * N, INPUT_DIM)), ent_all, tim_all],
        axis=1)

    WZH = jnp.concatenate([ezW_ref[...], ehW_ref[...]], axis=1)
    XW_all = _mm(h_all, WZH)
    V_all = jnp.concatenate([dinv, dinv, dinv], axis=0) * XW_all

    V = jnp.concatenate(
        [V_all[t * N:(t + 1) * N] for t in range(PERIODS)], axis=1)
    U = _colsum_contract(S, V)

    bzh = jnp.concatenate([bias(_B_ECZ), bias(_B_ECH)], axis=1)
    zeros_hh = jnp.zeros((HIDDEN, HIDDEN), jnp.float32)
    BD = jnp.concatenate([
        jnp.concatenate([elzW_ref[:HIDDEN], zeros_hh], axis=1),
        jnp.concatenate([zeros_hh, elhW_ref[:HIDDEN]], axis=1)], axis=0)
    blz = bias(_B_ELZ)
    blh = bias(_B_ELH)

    Hacc = jnp.zeros((N, HIDDEN), jnp.float32)
    for t in range(PERIODS):
        xw_t = XW_all[t * N:(t + 1) * N]
        c_t = dinv * U[:, t * 128:(t + 1) * 128] + dinv2 * xw_t + bzh
        G = _mm(c_t, BD)
        Z = jax.nn.sigmoid(G[:, :HIDDEN] + blz)
        Ht = jnp.tanh(G[:, HIDDEN:] + blh)
        Hacc = Hacc + probs[0, t] * ((1.0 - Z) * Ht)

    enc = jax.nn.relu(Hacc)
    mulvW = jnp.concatenate([muW_ref[...], lvW_ref[...]], axis=1)
    mulvb = jnp.concatenate(
        [bias(_B_MU, LATENT), bias(_B_LV, LATENT)], axis=1)
    mulv = _mm(enc, mulvW) + mulvb
    mu = mulv[:, :LATENT]
    lv = mulv[:, LATENT:]
    mu_ref[...] = mu
    lv_ref[...] = lv
    z = mu + eps_ref[...] * jnp.exp(0.5 * lv)
    dh = _mm(z, decW_ref[...]) + bias(_B_DEC)

    WZH_d = jnp.concatenate([dzW_ref[...], dhW_ref[...]], axis=1)
    xw_d = _mm(dh, WZH_d)
    u_d = _colsum_contract(S, dinv * xw_d)
    bzh_d = jnp.concatenate(
        [bias(_B_DCZ, INPUT_DIM), bias(_B_DCH, INPUT_DIM)], axis=1)
    c_d = dinv * u_d + dinv2 * xw_d + bzh_d
    zeros_ii = jnp.zeros((INPUT_DIM, INPUT_DIM), jnp.float32)
    BD_d = jnp.concatenate([
        jnp.concatenate([dlzW_ref[:INPUT_DIM], zeros_ii], axis=1),
        jnp.concatenate([zeros_ii, dlhW_ref[:INPUT_DIM]], axis=1)], axis=0)
    G_d = _mm(c_d, BD_d)
    Zd = jax.nn.sigmoid(G_d[:, :INPUT_DIM] + bias(_B_DLZ, INPUT_DIM))
    Htd = jnp.tanh(G_d[:, INPUT_DIM:] + bias(_B_DLH, INPUT_DIM))
    recon_ref[...] = jax.nn.relu((1.0 - Zd) * Htd)


def kernel(x, entity_emb, time_emb, num_nodes, params):
    p = params
    f32 = jnp.float32
    biases = jnp.concatenate([
        p['ent_b'], p['time_b'], p['e_conv_z_b'], p['e_lin_z_b'],
        p['e_conv_h_b'], p['e_lin_h_b'], p['mu_b'], p['lv_b'], p['dec_b'],
        p['d_conv_z_b'], p['d_lin_z_b'], p['d_conv_h_b'], p['d_lin_h_b'],
        p['att']])[None, :]
    operands = [
        x, entity_emb, time_emb, jnp.asarray(_EPS),
        p['W_score'], p['A_score'],
        p['ent_W'], p['time_W'], p['e_conv_z_W'], p['e_conv_h_W'],
        p['e_lin_z_W'], p['e_lin_h_W'],
        p['mu_W'], p['lv_W'], p['dec_W'],
        p['d_conv_z_W'], p['d_conv_h_W'], p['d_lin_z_W'], p['d_lin_h_W'],
        biases,
    ]
    out_shape = (
        jax.ShapeDtypeStruct((N, INPUT_DIM), f32),
        jax.ShapeDtypeStruct((N, LATENT), f32),
        jax.ShapeDtypeStruct((N, LATENT), f32),
        jax.ShapeDtypeStruct((N, N), f32),
        jax.ShapeDtypeStruct((N, N), f32),
    )
    return pl.pallas_call(_fwd_kernel, out_shape=out_shape)(*operands)


# speedup vs baseline: 1.2942x; 1.2942x over previous
import jax
import jax.numpy as jnp
import numpy as _np
from jax.experimental import pallas as pl

N = 512
INPUT_DIM = 32
EMBED_DIM = 64
HIDDEN = 64
LATENT = 32
PERIODS = 3

_EPS = _np.asarray(
    jax.random.normal(jax.random.key(42), (N, LATENT), jnp.float32))

_M64_ENTA = 0
_M64_TIMA = 1536
_M64_ENTW = 3072
_M64_TIMW = 3136
_M64_EZW = 3200
_M64_EHW = 3360
_M64_ELZ = 3520
_M64_ELH = 3648
_M64_DECW = 3776
_M64_ROWS = 3808

_M32_X = 0
_M32_EPS = 1536
_M32_MUW = 2048
_M32_LVW = 2112
_M32_DZW = 2176
_M32_DHW = 2240
_M32_DLZ = 2304
_M32_DLH = 2368
_M32_ROWS = 2432

_B_ENT, _B_TIM, _B_ECZ, _B_ELZ, _B_ECH, _B_ELH = 0, 64, 128, 192, 256, 320
_B_MU, _B_LV, _B_DEC = 384, 416, 448
_B_DCZ, _B_DLZ, _B_DCH, _B_DLH, _B_ATT = 512, 544, 576, 608, 640


def _colsum_contract(a, b):
    return jax.lax.dot_general(
        a, b, (((0,), (0,)), ((), ())), preferred_element_type=jnp.float32)


def _mm(a, b):
    return jnp.dot(a, b, preferred_element_type=jnp.float32)


def _fwd_kernel(
    ws_ref, as_ref, m64_ref, m32_ref, bias_ref,
    recon_ref, mu_ref, lv_ref, w_ref, a_ref,
):
    def bias(off, width=HIDDEN):
        return bias_ref[0:1, off:off + width]

    ri = jax.lax.broadcasted_iota(jnp.int32, (N, N), 0)
    ci = jax.lax.broadcasted_iota(jnp.int32, (N, N), 1)
    W = jnp.where(ri == ci, 0.0, jax.nn.sigmoid(ws_ref[...]))
    A = jax.nn.sigmoid(as_ref[...])
    w_ref[...] = W
    a_ref[...] = A
    S = W + A

    ones = jnp.ones((N, 1), jnp.float32)
    deg = _colsum_contract(S, ones) + 1.0
    dinv = jax.lax.rsqrt(deg)
    dinv2 = dinv * dinv

    probs = jax.nn.softmax(bias(_B_ATT, PERIODS), axis=-1)

    ent_all = jax.nn.relu(
        _mm(m64_ref[_M64_ENTA:_M64_ENTA + PERIODS * N],
            m64_ref[_M64_ENTW:_M64_ENTW + EMBED_DIM]) + bias(_B_ENT))
    tim_all = jax.nn.relu(
        _mm(m64_ref[_M64_TIMA:_M64_TIMA + PERIODS * N],
            m64_ref[_M64_TIMW:_M64_TIMW + EMBED_DIM]) + bias(_B_TIM))
    h_all = jnp.concatenate(
        [m32_ref[_M32_X:_M32_X + PERIODS * N], ent_all, tim_all],
        axis=1)

    cin = INPUT_DIM + 2 * HIDDEN
    WZH = jnp.concatenate([
        m64_ref[_M64_EZW:_M64_EZW + cin],
        m64_ref[_M64_EHW:_M64_EHW + cin]], axis=1)
    XW_all = _mm(h_all, WZH)
    V_all = jnp.concatenate([dinv, dinv, dinv], axis=0) * XW_all

    V = jnp.concatenate(
        [V_all[t * N:(t + 1) * N] for t in range(PERIODS)], axis=1)
    U = _colsum_contract(S, V)

    bzh = jnp.concatenate([bias(_B_ECZ), bias(_B_ECH)], axis=1)
    zeros_hh = jnp.zeros((HIDDEN, HIDDEN), jnp.float32)
    BD = jnp.concatenate([
        jnp.concatenate([m64_ref[_M64_ELZ:_M64_ELZ + HIDDEN], zeros_hh],
                        axis=1),
        jnp.concatenate([zeros_hh, m64_ref[_M64_ELH:_M64_ELH + HIDDEN]],
                        axis=1)], axis=0)
    blz = bias(_B_ELZ)
    blh = bias(_B_ELH)

    Hacc = jnp.zeros((N, HIDDEN), jnp.float32)
    for t in range(PERIODS):
        xw_t = XW_all[t * N:(t + 1) * N]
        c_t = dinv * U[:, t * 128:(t + 1) * 128] + dinv2 * xw_t + bzh
        G = _mm(c_t, BD)
        Z = jax.nn.sigmoid(G[:, :HIDDEN] + blz)
        Ht = jnp.tanh(G[:, HIDDEN:] + blh)
        Hacc = Hacc + probs[0, t] * ((1.0 - Z) * Ht)

    enc = jax.nn.relu(Hacc)
    mulvW = jnp.concatenate([
        m32_ref[_M32_MUW:_M32_MUW + HIDDEN],
        m32_ref[_M32_LVW:_M32_LVW + HIDDEN]], axis=1)
    mulvb = jnp.concatenate(
        [bias(_B_MU, LATENT), bias(_B_LV, LATENT)], axis=1)
    mulv = _mm(enc, mulvW) + mulvb
    mu = mulv[:, :LATENT]
    lv = mulv[:, LATENT:]
    mu_ref[...] = mu
    lv_ref[...] = lv
    z = mu + m32_ref[_M32_EPS:_M32_EPS + N] * jnp.exp(0.5 * lv)
    dh = _mm(z, m64_ref[_M64_DECW:_M64_DECW + LATENT]) + bias(_B_DEC)

    WZH_d = jnp.concatenate([
        m32_ref[_M32_DZW:_M32_DZW + HIDDEN],
        m32_ref[_M32_DHW:_M32_DHW + HIDDEN]], axis=1)
    xw_d = _mm(dh, WZH_d)
    u_d = _colsum_contract(S, dinv * xw_d)
    bzh_d = jnp.concatenate(
        [bias(_B_DCZ, INPUT_DIM), bias(_B_DCH, INPUT_DIM)], axis=1)
    c_d = dinv * u_d + dinv2 * xw_d + bzh_d
    zeros_ii = jnp.zeros((INPUT_DIM, INPUT_DIM), jnp.float32)
    BD_d = jnp.concatenate([
        jnp.concatenate([m32_ref[_M32_DLZ:_M32_DLZ + INPUT_DIM], zeros_ii],
                        axis=1),
        jnp.concatenate([zeros_ii, m32_ref[_M32_DLH:_M32_DLH + INPUT_DIM]],
                        axis=1)], axis=0)
    G_d = _mm(c_d, BD_d)
    Zd = jax.nn.sigmoid(G_d[:, :INPUT_DIM] + bias(_B_DLZ, INPUT_DIM))
    Htd = jnp.tanh(G_d[:, INPUT_DIM:] + bias(_B_DLH, INPUT_DIM))
    recon_ref[...] = jax.nn.relu((1.0 - Zd) * Htd)


def kernel(x, entity_emb, time_emb, num_nodes, params):
    p = params
    f32 = jnp.float32
    m64 = jnp.concatenate([
        jnp.reshape(entity_emb, (PERIODS * N, EMBED_DIM)),
        jnp.reshape(time_emb, (PERIODS * N, EMBED_DIM)),
        p['ent_W'], p['time_W'], p['e_conv_z_W'], p['e_conv_h_W'],
        p['e_lin_z_W'], p['e_lin_h_W'], p['dec_W']], axis=0)
    m32 = jnp.concatenate([
        jnp.reshape(x, (PERIODS * N, INPUT_DIM)), jnp.asarray(_EPS),
        p['mu_W'], p['lv_W'], p['d_conv_z_W'], p['d_conv_h_W'],
        p['d_lin_z_W'], p['d_lin_h_W']], axis=0)
    biases = jnp.concatenate([
        p['ent_b'], p['time_b'], p['e_conv_z_b'], p['e_lin_z_b'],
        p['e_conv_h_b'], p['e_lin_h_b'], p['mu_b'], p['lv_b'], p['dec_b'],
        p['d_conv_z_b'], p['d_lin_z_b'], p['d_conv_h_b'], p['d_lin_h_b'],
        p['att']])[None, :]
    operands = [p['W_score'], p['A_score'], m64, m32, biases]
    out_shape = (
        jax.ShapeDtypeStruct((N, INPUT_DIM), f32),
        jax.ShapeDtypeStruct((N, LATENT), f32),
        jax.ShapeDtypeStruct((N, LATENT), f32),
        jax.ShapeDtypeStruct((N, N), f32),
        jax.ShapeDtypeStruct((N, N), f32),
    )
    return pl.pallas_call(_fwd_kernel, out_shape=out_shape)(*operands)


# speedup vs baseline: 1.3516x; 1.0443x over previous
import jax
import jax.numpy as jnp
import numpy as _np
from jax.experimental import pallas as pl

N = 512
INPUT_DIM = 32
EMBED_DIM = 64
HIDDEN = 64
LATENT = 32
PERIODS = 3

_EPS = _np.asarray(
    jax.random.normal(jax.random.key(42), (N, LATENT), jnp.float32))

_M64_ENTA = 0
_M64_TIMA = 1536
_M64_ENTW = 3072
_M64_TIMW = 3136
_M64_EZW = 3200
_M64_EHW = 3360
_M64_ELZ = 3520
_M64_ELH = 3648
_M64_DECW = 3776
_M64_BIAS = 3808
_M64_ROWS = 3819

_M32_X = 0
_M32_EPS = 1536
_M32_MUW = 2048
_M32_LVW = 2112
_M32_DZW = 2176
_M32_DHW = 2240
_M32_DLZ = 2304
_M32_DLH = 2368
_M32_ROWS = 2432

_B_ENT, _B_TIM, _B_ECZ, _B_ELZ, _B_ECH, _B_ELH = 0, 64, 128, 192, 256, 320
_B_MU, _B_LV, _B_DEC = 384, 416, 448
_B_DCZ, _B_DLZ, _B_DCH, _B_DLH, _B_ATT = 512, 544, 576, 608, 640


def _colsum_contract(a, b):
    return jax.lax.dot_general(
        a, b, (((0,), (0,)), ((), ())), preferred_element_type=jnp.float32)


def _mm(a, b):
    return jnp.dot(a, b, preferred_element_type=jnp.float32)


def _fwd_kernel(
    ws_ref, as_ref, m64_ref, m32_ref,
    recon_ref, mu_ref, lv_ref, w_ref, a_ref,
):
    def bias(off, width=HIDDEN):
        row = _M64_BIAS + off // 64
        lane = off % 64
        return m64_ref[row:row + 1, lane:lane + width]

    ri = jax.lax.broadcasted_iota(jnp.int32, (N, N), 0)
    ci = jax.lax.broadcasted_iota(jnp.int32, (N, N), 1)
    W = jnp.where(ri == ci, 0.0, jax.nn.sigmoid(ws_ref[...]))
    A = jax.nn.sigmoid(as_ref[...])
    w_ref[...] = W
    a_ref[...] = A
    S = W + A

    ones = jnp.ones((N, 1), jnp.float32)
    deg = _colsum_contract(S, ones) + 1.0
    dinv = jax.lax.rsqrt(deg)
    dinv2 = dinv * dinv

    probs = jax.nn.softmax(bias(_B_ATT, PERIODS), axis=-1)

    ent_all = jax.nn.relu(
        _mm(m64_ref[_M64_ENTA:_M64_ENTA + PERIODS * N],
            m64_ref[_M64_ENTW:_M64_ENTW + EMBED_DIM]) + bias(_B_ENT))
    tim_all = jax.nn.relu(
        _mm(m64_ref[_M64_TIMA:_M64_TIMA + PERIODS * N],
            m64_ref[_M64_TIMW:_M64_TIMW + EMBED_DIM]) + bias(_B_TIM))
    h_all = jnp.concatenate(
        [m32_ref[_M32_X:_M32_X + PERIODS * N], ent_all, tim_all],
        axis=1)

    cin = INPUT_DIM + 2 * HIDDEN
    WZH = jnp.concatenate([
        m64_ref[_M64_EZW:_M64_EZW + cin],
        m64_ref[_M64_EHW:_M64_EHW + cin]], axis=1)
    XW_all = _mm(h_all, WZH)
    V_all = jnp.concatenate([dinv, dinv, dinv], axis=0) * XW_all

    V = jnp.concatenate(
        [V_all[t * N:(t + 1) * N] for t in range(PERIODS)], axis=1)
    U = _colsum_contract(S, V)

    bzh = jnp.concatenate([bias(_B_ECZ), bias(_B_ECH)], axis=1)
    zeros_hh = jnp.zeros((HIDDEN, HIDDEN), jnp.float32)
    BD = jnp.concatenate([
        jnp.concatenate([m64_ref[_M64_ELZ:_M64_ELZ + HIDDEN], zeros_hh],
                        axis=1),
        jnp.concatenate([zeros_hh, m64_ref[_M64_ELH:_M64_ELH + HIDDEN]],
                        axis=1)], axis=0)
    blz = bias(_B_ELZ)
    blh = bias(_B_ELH)

    Hacc = jnp.zeros((N, HIDDEN), jnp.float32)
    for t in range(PERIODS):
        xw_t = XW_all[t * N:(t + 1) * N]
        c_t = dinv * U[:, t * 128:(t + 1) * 128] + dinv2 * xw_t + bzh
        G = _mm(c_t, BD)
        Z = jax.nn.sigmoid(G[:, :HIDDEN] + blz)
        Ht = jnp.tanh(G[:, HIDDEN:] + blh)
        Hacc = Hacc + probs[0, t] * ((1.0 - Z) * Ht)

    enc = jax.nn.relu(Hacc)
    mulvW = jnp.concatenate([
        m32_ref[_M32_MUW:_M32_MUW + HIDDEN],
        m32_ref[_M32_LVW:_M32_LVW + HIDDEN]], axis=1)
    mulvb = jnp.concatenate(
        [bias(_B_MU, LATENT), bias(_B_LV, LATENT)], axis=1)
    mulv = _mm(enc, mulvW) + mulvb
    mu = mulv[:, :LATENT]
    lv = mulv[:, LATENT:]
    mu_ref[...] = mu
    lv_ref[...] = lv
    z = mu + m32_ref[_M32_EPS:_M32_EPS + N] * jnp.exp(0.5 * lv)
    dh = _mm(z, m64_ref[_M64_DECW:_M64_DECW + LATENT]) + bias(_B_DEC)

    WZH_d = jnp.concatenate([
        m32_ref[_M32_DZW:_M32_DZW + HIDDEN],
        m32_ref[_M32_DHW:_M32_DHW + HIDDEN]], axis=1)
    xw_d = _mm(dh, WZH_d)
    u_d = _colsum_contract(S, dinv * xw_d)
    bzh_d = jnp.concatenate(
        [bias(_B_DCZ, INPUT_DIM), bias(_B_DCH, INPUT_DIM)], axis=1)
    c_d = dinv * u_d + dinv2 * xw_d + bzh_d
    zeros_ii = jnp.zeros((INPUT_DIM, INPUT_DIM), jnp.float32)
    BD_d = jnp.concatenate([
        jnp.concatenate([m32_ref[_M32_DLZ:_M32_DLZ + INPUT_DIM], zeros_ii],
                        axis=1),
        jnp.concatenate([zeros_ii, m32_ref[_M32_DLH:_M32_DLH + INPUT_DIM]],
                        axis=1)], axis=0)
    G_d = _mm(c_d, BD_d)
    Zd = jax.nn.sigmoid(G_d[:, :INPUT_DIM] + bias(_B_DLZ, INPUT_DIM))
    Htd = jnp.tanh(G_d[:, INPUT_DIM:] + bias(_B_DLH, INPUT_DIM))
    recon_ref[...] = jax.nn.relu((1.0 - Zd) * Htd)


def kernel(x, entity_emb, time_emb, num_nodes, params):
    p = params
    f32 = jnp.float32
    biases = jnp.reshape(jnp.pad(jnp.concatenate([
        p['ent_b'], p['time_b'], p['e_conv_z_b'], p['e_lin_z_b'],
        p['e_conv_h_b'], p['e_lin_h_b'], p['mu_b'], p['lv_b'], p['dec_b'],
        p['d_conv_z_b'], p['d_lin_z_b'], p['d_conv_h_b'], p['d_lin_h_b'],
        p['att']]), (0, 61)), (11, 64))
    m64 = jnp.concatenate([
        jnp.reshape(entity_emb, (PERIODS * N, EMBED_DIM)),
        jnp.reshape(time_emb, (PERIODS * N, EMBED_DIM)),
        p['ent_W'], p['time_W'], p['e_conv_z_W'], p['e_conv_h_W'],
        p['e_lin_z_W'], p['e_lin_h_W'], p['dec_W'], biases], axis=0)
    m32 = jnp.concatenate([
        jnp.reshape(x, (PERIODS * N, INPUT_DIM)), jnp.asarray(_EPS),
        p['mu_W'], p['lv_W'], p['d_conv_z_W'], p['d_conv_h_W'],
        p['d_lin_z_W'], p['d_lin_h_W']], axis=0)
    operands = [p['W_score'], p['A_score'], m64, m32]
    out_shape = (
        jax.ShapeDtypeStruct((N, INPUT_DIM), f32),
        jax.ShapeDtypeStruct((N, LATENT), f32),
        jax.ShapeDtypeStruct((N, LATENT), f32),
        jax.ShapeDtypeStruct((N, N), f32),
        jax.ShapeDtypeStruct((N, N), f32),
    )
    return pl.pallas_call(_fwd_kernel, out_shape=out_shape)(*operands)
